# Initial kernel scaffold; baseline (speedup 1.0000x reference)
import jax
import jax.numpy as jnp
from jax.experimental import pallas as pl


def kernel(x, edge_index, W0, b0, W1, b1, W2, b2, bn0_g, bn0_b, bn0_m, bn0_v, bn1_g, bn1_b, bn1_m, bn1_v):
    raise NotImplementedError("write your pallas kernel here")


# speedup vs baseline: 11.4219x; 11.4219x over previous
import jax
import jax.numpy as jnp
from jax import lax
from jax.experimental import pallas as pl
from jax.experimental.pallas import tpu as pltpu
from jax.experimental.pallas import tpu_sc as plsc

N = 10000
E = 320000
D = 128
EPS = 1e-5

NC = 2
NS = 16
NW = NC * NS
EPW = E // NW
EB = 80
NCHUNK = EPW // EB
RCH = N // EB
RNDS = -(-RCH // NS)

_MESH = plsc.VectorSubcoreMesh(
    core_axis_name="c", subcore_axis_name="s", num_cores=NC, num_subcores=NS
)


def _sc_agg_body(g_hbm, src_hbm, dst_hbm, out_hbm,
                 acc, src_v, dst_v, rows_v, sem):
  c = lax.axis_index("c")
  s = lax.axis_index("s")
  wid = s * NC + c

  for j in range(RNDS):
    k = s + NS * j

    @pl.when(k < RCH)
    def _():
      pltpu.sync_copy(g_hbm.at[pl.ds(k * EB, EB)], rows_v)
      pltpu.sync_copy(rows_v, acc.at[pl.ds(k * EB, EB)])

  plsc.subcore_barrier()

  base = wid * EPW

  def chunk(i, carry):
    off = base + i * EB
    pltpu.sync_copy(src_hbm.at[pl.ds(off, EB)], src_v)
    pltpu.sync_copy(dst_hbm.at[pl.ds(off, EB)], dst_v)
    pltpu.async_copy(g_hbm.at[src_v], rows_v, sem).wait()
    pltpu.sync_copy(rows_v, acc.at[dst_v], add=True)
    return carry

  lax.fori_loop(0, NCHUNK, chunk, 0)
  plsc.subcore_barrier()

  for j in range(RNDS):
    k = s + NS * j

    @pl.when(k < RCH)
    def _():
      pltpu.sync_copy(acc.at[pl.ds(k * EB, EB)], rows_v)
      pltpu.sync_copy(rows_v, out_hbm.at[c, pl.ds(k * EB, EB)])


_sc_agg = pl.kernel(
    _sc_agg_body,
    out_type=jax.ShapeDtypeStruct((NC, N, D), jnp.float32),
    mesh=_MESH,
    scratch_types=[
        pltpu.VMEM_SHARED((N, D), jnp.float32),
        pltpu.VMEM((EB,), jnp.int32),
        pltpu.VMEM((EB,), jnp.int32),
        pltpu.VMEM((EB, D), jnp.float32),
        pltpu.SemaphoreType.DMA,
    ],
    name="sc_gcn_aggregate",
)


def _sc_deg_body(dst_hbm, out_hbm, acc, dst_v, ones_v, stage_v):
  c = lax.axis_index("c")
  s = lax.axis_index("s")
  wid = s * NC + c

  for j in range(EB // 16):
    ones_v[pl.ds(j * 16, 16)] = jnp.ones((16,), jnp.float32)

  for j in range(RNDS):
    k = s + NS * j

    @pl.when(k < RCH)
    def _():
      pltpu.sync_copy(ones_v, acc.at[pl.ds(k * EB, EB)])

  plsc.subcore_barrier()

  base = wid * EPW

  def chunk(i, carry):
    pltpu.sync_copy(dst_hbm.at[pl.ds(base + i * EB, EB)], dst_v)
    pltpu.sync_copy(ones_v, acc.at[dst_v], add=True)
    return carry

  lax.fori_loop(0, NCHUNK, chunk, 0)
  plsc.subcore_barrier()

  for j in range(RNDS):
    k = s + NS * j

    @pl.when(k < RCH)
    def _():
      pltpu.sync_copy(acc.at[pl.ds(k * EB, EB)], stage_v)
      pltpu.sync_copy(stage_v, out_hbm.at[pl.ds(c * N + k * EB, EB)])


_sc_deg = pl.kernel(
    _sc_deg_body,
    out_type=jax.ShapeDtypeStruct((NC * N,), jnp.float32),
    mesh=_MESH,
    scratch_types=[
        pltpu.VMEM_SHARED((N,), jnp.float32),
        pltpu.VMEM((EB,), jnp.int32),
        pltpu.VMEM((EB,), jnp.float32),
        pltpu.VMEM((EB,), jnp.float32),
    ],
    name="sc_gcn_degree",
)


RBK = 1000
GRID = N // RBK


def _dis(pdegT):
  return lax.rsqrt(pdegT[:, 0:1] + pdegT[:, 1:2] - 1.0)


def _tc0_body(x_ref, w_ref, pdegT_ref, g_ref):
  dis = _dis(pdegT_ref[...])
  g_ref[...] = jnp.dot(x_ref[...], w_ref[...],
                       preferred_element_type=jnp.float32) * dis


def _tcA_body(p_ref, g_ref, pdegT_ref, w_ref, s_ref, t_ref, h_ref, gn_ref):
  dis = _dis(pdegT_ref[...])
  z = (p_ref[0] + p_ref[1] - g_ref[...]) * dis
  h = jnp.maximum(z * s_ref[...] + t_ref[...], 0.0)
  h_ref[...] = h
  gn_ref[...] = jnp.dot(h, w_ref[...],
                        preferred_element_type=jnp.float32) * dis


def _tcB_body(p_ref, g_ref, pdegT_ref, w_ref, s_ref, t_ref, res_ref, gn_ref):
  dis = _dis(pdegT_ref[...])
  z = (p_ref[0] + p_ref[1] - g_ref[...]) * dis
  h = jnp.maximum(z * s_ref[...] + t_ref[...], 0.0) + res_ref[...]
  gn_ref[...] = jnp.dot(h, w_ref[...],
                        preferred_element_type=jnp.float32) * dis


def _tcF_body(p_ref, g_ref, pdegT_ref, b_ref, o_ref):
  dis = _dis(pdegT_ref[...])
  o_ref[...] = (p_ref[0] + p_ref[1] - g_ref[...]) * dis + b_ref[...]


_row_spec = pl.BlockSpec((RBK, D), lambda i: (i, 0))
_p_spec = pl.BlockSpec((NC, RBK, D), lambda i: (0, i, 0))
_w_spec = pl.BlockSpec((D, D), lambda i: (0, 0))
_v_spec = pl.BlockSpec((1, D), lambda i: (0, 0))
_pdegT_spec = pl.BlockSpec((RBK, NC), lambda i: (i, 0))
_row_out = jax.ShapeDtypeStruct((N, D), jnp.float32)

_tc0 = pl.pallas_call(
    _tc0_body, grid=(GRID,),
    in_specs=[_row_spec, _w_spec, _pdegT_spec],
    out_specs=_row_spec, out_shape=_row_out)

_tcA = pl.pallas_call(
    _tcA_body, grid=(GRID,),
    in_specs=[_p_spec, _row_spec, _pdegT_spec, _w_spec, _v_spec, _v_spec],
    out_specs=(_row_spec, _row_spec), out_shape=(_row_out, _row_out))

_tcB = pl.pallas_call(
    _tcB_body, grid=(GRID,),
    in_specs=[_p_spec, _row_spec, _pdegT_spec, _w_spec, _v_spec, _v_spec,
              _row_spec],
    out_specs=_row_spec, out_shape=_row_out)

_tcF = pl.pallas_call(
    _tcF_body, grid=(GRID,),
    in_specs=[_p_spec, _row_spec, _pdegT_spec, _v_spec],
    out_specs=_row_spec, out_shape=_row_out)


def kernel(x, edge_index, W0, b0, W1, b1, W2, b2,
           bn0_g, bn0_b, bn0_m, bn0_v, bn1_g, bn1_b, bn1_m, bn1_v):
  src = edge_index[0].astype(jnp.int32)
  dst = edge_index[1].astype(jnp.int32)

  S0 = (bn0_g * lax.rsqrt(bn0_v + EPS)).reshape(1, D)
  T0 = ((b0 - bn0_m) * S0[0] + bn0_b).reshape(1, D)
  S1 = (bn1_g * lax.rsqrt(bn1_v + EPS)).reshape(1, D)
  T1 = ((b1 - bn1_m) * S1[0] + bn1_b).reshape(1, D)
  b2r = b2.reshape(1, D)

  pdegT = _sc_deg(dst).reshape(NC, N).T

  g0 = _tc0(x, W0, pdegT)
  p0 = _sc_agg(g0, src, dst)
  h0, g1 = _tcA(p0, g0, pdegT, W1, S0, T0)
  p1 = _sc_agg(g1, src, dst)
  g2 = _tcB(p1, g1, pdegT, W2, S1, T1, h0)
  p2 = _sc_agg(g2, src, dst)
  return _tcF(p2, g2, pdegT, b2r)


# speedup vs baseline: 27.8639x; 2.4395x over previous
import jax
import jax.numpy as jnp
from jax import lax
from jax.experimental import pallas as pl
from jax.experimental.pallas import tpu as pltpu
from jax.experimental.pallas import tpu_sc as plsc

N = 10000
E = 320000
D = 128
EPS = 1e-5

NC = 2
NS = 16
NW = NC * NS
EPW = E // NW
EB = 80
NCHUNK = EPW // EB
RCH = N // EB
RNDS = -(-RCH // NS)

_MESH = plsc.VectorSubcoreMesh(
    core_axis_name="c", subcore_axis_name="s", num_cores=NC, num_subcores=NS
)


NSLOT = 3
PRO = 2
NRND = (NCHUNK - PRO) // 3


def _sc_agg_body(g_hbm, src_hbm, dst_hbm, out_hbm,
                 acc, idx_s, idxd, bufs, semi, semg, sems, semw):
  c = lax.axis_index("c")
  s = lax.axis_index("s")
  wid = s * NC + c

  def fire(i, sl):
    pltpu.async_copy(dst_hbm.at[wid, i], idxd.at[sl], semi.at[sl])
    off = pl.multiple_of(i * EB, EB)
    pltpu.async_copy(g_hbm.at[idx_s.at[pl.ds(off, EB)]], bufs.at[sl],
                     semg.at[sl])

  def gdrain(i, sl):
    pltpu.make_async_copy(dst_hbm.at[wid, i], idxd.at[sl], semi.at[sl]).wait()
    off = pl.multiple_of(i * EB, EB)
    pltpu.make_async_copy(g_hbm.at[idx_s.at[pl.ds(off, EB)]], bufs.at[sl],
                          semg.at[sl]).wait()

  def sfire(sl):
    pltpu.async_copy(bufs.at[sl], acc.at[idxd.at[sl]], sems.at[sl], add=True)

  def sdrain(sl):
    pltpu.make_async_copy(bufs.at[sl], acc.at[idxd.at[sl]],
                          sems.at[sl]).wait()

  pltpu.sync_copy(src_hbm.at[wid], idx_s)

  for j in range(RNDS):
    k = s + NS * j

    @pl.when(k < RCH)
    def _():
      pltpu.async_copy(g_hbm.at[pl.ds(k * EB, EB)],
                       acc.at[pl.ds(k * EB, EB)], semw)

  fire(0, 0)
  fire(1, 1)

  for j in range(RNDS):
    k = s + NS * j

    @pl.when(k < RCH)
    def _():
      pltpu.make_async_copy(g_hbm.at[pl.ds(k * EB, EB)],
                            acc.at[pl.ds(k * EB, EB)], semw).wait()

  plsc.subcore_barrier()

  gdrain(0, 0)
  sfire(0)
  fire(2, 2)
  gdrain(1, 1)
  sfire(1)
  sdrain(0)
  fire(3, 0)

  def rounds(rr, carry):
    i0 = 3 * rr + PRO
    for j in range(3):
      i = i0 + j
      sl = (PRO + j) % 3
      slp = (PRO + j - 1) % 3
      gdrain(i, sl)
      sfire(sl)
      sdrain(slp)

      @pl.when(i + PRO < NCHUNK)
      def _():
        fire(i + PRO, slp)

    return carry

  lax.fori_loop(0, NRND, rounds, 0)
  sdrain((NCHUNK - 1) % 3)
  plsc.subcore_barrier()

  for j in range(RNDS):
    k = s + NS * j

    @pl.when(k < RCH)
    def _():
      pltpu.async_copy(acc.at[pl.ds(k * EB, EB)],
                       out_hbm.at[c, pl.ds(k * EB, EB)], semw)

  for j in range(RNDS):
    k = s + NS * j

    @pl.when(k < RCH)
    def _():
      pltpu.make_async_copy(acc.at[pl.ds(k * EB, EB)],
                            out_hbm.at[c, pl.ds(k * EB, EB)], semw).wait()


_sc_agg = pl.kernel(
    _sc_agg_body,
    out_type=jax.ShapeDtypeStruct((NC, N, D), jnp.float32),
    mesh=_MESH,
    scratch_types=[
        pltpu.VMEM_SHARED((N, D), jnp.float32),
        pltpu.VMEM((EPW,), jnp.int32),
        pltpu.VMEM((NSLOT, EB), jnp.int32),
        pltpu.VMEM((NSLOT, EB, D), jnp.float32),
        pltpu.SemaphoreType.DMA((NSLOT,)),
        pltpu.SemaphoreType.DMA((NSLOT,)),
        pltpu.SemaphoreType.DMA((NSLOT,)),
        pltpu.SemaphoreType.DMA,
    ],
    name="sc_gcn_aggregate",
)


def _sc_deg_body(dst_hbm, out_hbm, acc, dst_v, ones_v, stage_v):
  c = lax.axis_index("c")
  s = lax.axis_index("s")
  wid = s * NC + c

  for j in range(EB // 16):
    ones_v[pl.ds(j * 16, 16)] = jnp.ones((16,), jnp.float32)

  for j in range(RNDS):
    k = s + NS * j

    @pl.when(k < RCH)
    def _():
      pltpu.sync_copy(ones_v, acc.at[pl.ds(k * EB, EB)])

  plsc.subcore_barrier()

  base = wid * EPW

  def chunk(i, carry):
    pltpu.sync_copy(dst_hbm.at[pl.ds(base + i * EB, EB)], dst_v)
    pltpu.sync_copy(ones_v, acc.at[dst_v], add=True)
    return carry

  lax.fori_loop(0, NCHUNK, chunk, 0)
  plsc.subcore_barrier()

  for j in range(RNDS):
    k = s + NS * j

    @pl.when(k < RCH)
    def _():
      pltpu.sync_copy(acc.at[pl.ds(k * EB, EB)], stage_v)
      pltpu.sync_copy(stage_v, out_hbm.at[pl.ds(c * N + k * EB, EB)])


_sc_deg = pl.kernel(
    _sc_deg_body,
    out_type=jax.ShapeDtypeStruct((NC * N,), jnp.float32),
    mesh=_MESH,
    scratch_types=[
        pltpu.VMEM_SHARED((N,), jnp.float32),
        pltpu.VMEM((EB,), jnp.int32),
        pltpu.VMEM((EB,), jnp.float32),
        pltpu.VMEM((EB,), jnp.float32),
    ],
    name="sc_gcn_degree",
)


RBK = 1000
GRID = N // RBK


def _dis(pdegT):
  return lax.rsqrt(pdegT[:, 0:1] + pdegT[:, 1:2] - 1.0)


def _tc0_body(x_ref, w_ref, pdegT_ref, g_ref):
  dis = _dis(pdegT_ref[...])
  g_ref[...] = jnp.dot(x_ref[...], w_ref[...],
                       preferred_element_type=jnp.float32) * dis


def _tcA_body(p_ref, g_ref, pdegT_ref, w_ref, s_ref, t_ref, h_ref, gn_ref):
  dis = _dis(pdegT_ref[...])
  z = (p_ref[0] + p_ref[1] - g_ref[...]) * dis
  h = jnp.maximum(z * s_ref[...] + t_ref[...], 0.0)
  h_ref[...] = h
  gn_ref[...] = jnp.dot(h, w_ref[...],
                        preferred_element_type=jnp.float32) * dis


def _tcB_body(p_ref, g_ref, pdegT_ref, w_ref, s_ref, t_ref, res_ref, gn_ref):
  dis = _dis(pdegT_ref[...])
  z = (p_ref[0] + p_ref[1] - g_ref[...]) * dis
  h = jnp.maximum(z * s_ref[...] + t_ref[...], 0.0) + res_ref[...]
  gn_ref[...] = jnp.dot(h, w_ref[...],
                        preferred_element_type=jnp.float32) * dis


def _tcF_body(p_ref, g_ref, pdegT_ref, b_ref, o_ref):
  dis = _dis(pdegT_ref[...])
  o_ref[...] = (p_ref[0] + p_ref[1] - g_ref[...]) * dis + b_ref[...]


_row_spec = pl.BlockSpec((RBK, D), lambda i: (i, 0))
_p_spec = pl.BlockSpec((NC, RBK, D), lambda i: (0, i, 0))
_w_spec = pl.BlockSpec((D, D), lambda i: (0, 0))
_v_spec = pl.BlockSpec((1, D), lambda i: (0, 0))
_pdegT_spec = pl.BlockSpec((RBK, NC), lambda i: (i, 0))
_row_out = jax.ShapeDtypeStruct((N, D), jnp.float32)

_tc0 = pl.pallas_call(
    _tc0_body, grid=(GRID,),
    in_specs=[_row_spec, _w_spec, _pdegT_spec],
    out_specs=_row_spec, out_shape=_row_out)

_tcA = pl.pallas_call(
    _tcA_body, grid=(GRID,),
    in_specs=[_p_spec, _row_spec, _pdegT_spec, _w_spec, _v_spec, _v_spec],
    out_specs=(_row_spec, _row_spec), out_shape=(_row_out, _row_out))

_tcB = pl.pallas_call(
    _tcB_body, grid=(GRID,),
    in_specs=[_p_spec, _row_spec, _pdegT_spec, _w_spec, _v_spec, _v_spec,
              _row_spec],
    out_specs=_row_spec, out_shape=_row_out)

_tcF = pl.pallas_call(
    _tcF_body, grid=(GRID,),
    in_specs=[_p_spec, _row_spec, _pdegT_spec, _v_spec],
    out_specs=_row_spec, out_shape=_row_out)


def kernel(x, edge_index, W0, b0, W1, b1, W2, b2,
           bn0_g, bn0_b, bn0_m, bn0_v, bn1_g, bn1_b, bn1_m, bn1_v):
  src = edge_index[0].astype(jnp.int32)
  dst = edge_index[1].astype(jnp.int32)
  src3 = src.reshape(NW, EPW)
  dst3 = dst.reshape(NW, NCHUNK, EB)

  S0 = (bn0_g * lax.rsqrt(bn0_v + EPS)).reshape(1, D)
  T0 = ((b0 - bn0_m) * S0[0] + bn0_b).reshape(1, D)
  S1 = (bn1_g * lax.rsqrt(bn1_v + EPS)).reshape(1, D)
  T1 = ((b1 - bn1_m) * S1[0] + bn1_b).reshape(1, D)
  b2r = b2.reshape(1, D)

  pdegT = _sc_deg(dst).reshape(NC, N).T

  g0 = _tc0(x, W0, pdegT)
  p0 = _sc_agg(g0, src3, dst3)
  h0, g1 = _tcA(p0, g0, pdegT, W1, S0, T0)
  p1 = _sc_agg(g1, src3, dst3)
  g2 = _tcB(p1, g1, pdegT, W2, S1, T1, h0)
  p2 = _sc_agg(g2, src3, dst3)
  return _tcF(p2, g2, pdegT, b2r)


# speedup vs baseline: 32.7502x; 1.1754x over previous
import jax
import jax.numpy as jnp
from jax import lax
from jax.experimental import pallas as pl
from jax.experimental.pallas import tpu as pltpu
from jax.experimental.pallas import tpu_sc as plsc

N = 10000
E = 320000
D = 128
EPS = 1e-5

NC = 2
NS = 16
NW = NC * NS
EPW = E // NW
EB = 80
NCHUNK = EPW // EB
RCH = N // EB
RNDS = -(-RCH // NS)

_MESH = plsc.VectorSubcoreMesh(
    core_axis_name="c", subcore_axis_name="s", num_cores=NC, num_subcores=NS
)


NSLOT = 3
PRO = 2
NRND = (NCHUNK - PRO) // 3


def _sc_agg_body(g_hbm, src_hbm, dst_hbm, out_hbm,
                 acc, idx_s, idxd, bufs, semi, semg, sems, semw):
  c = lax.axis_index("c")
  s = lax.axis_index("s")
  wid = s * NC + c

  def fire(i, sl):
    pltpu.async_copy(dst_hbm.at[wid, i], idxd.at[sl], semi.at[sl])
    off = pl.multiple_of(i * EB, EB)
    pltpu.async_copy(g_hbm.at[idx_s.at[pl.ds(off, EB)]], bufs.at[sl],
                     semg.at[sl])

  def gdrain(i, sl):
    pltpu.make_async_copy(dst_hbm.at[wid, i], idxd.at[sl], semi.at[sl]).wait()
    off = pl.multiple_of(i * EB, EB)
    pltpu.make_async_copy(g_hbm.at[idx_s.at[pl.ds(off, EB)]], bufs.at[sl],
                          semg.at[sl]).wait()

  def sfire(sl):
    pltpu.async_copy(bufs.at[sl], acc.at[idxd.at[sl]], sems.at[sl], add=True)

  def sdrain(sl):
    pltpu.make_async_copy(bufs.at[sl], acc.at[idxd.at[sl]],
                          sems.at[sl]).wait()

  pltpu.sync_copy(src_hbm.at[wid], idx_s)

  for j in range(RNDS):
    k = s + NS * j

    @pl.when(k < RCH)
    def _():
      pltpu.async_copy(g_hbm.at[pl.ds(k * EB, EB)],
                       acc.at[pl.ds(k * EB, EB)], semw)

  fire(0, 0)
  fire(1, 1)

  for j in range(RNDS):
    k = s + NS * j

    @pl.when(k < RCH)
    def _():
      pltpu.make_async_copy(g_hbm.at[pl.ds(k * EB, EB)],
                            acc.at[pl.ds(k * EB, EB)], semw).wait()

  plsc.subcore_barrier()

  gdrain(0, 0)
  sfire(0)
  fire(2, 2)
  gdrain(1, 1)
  sfire(1)
  sdrain(0)
  fire(3, 0)

  def rounds(rr, carry):
    i0 = 3 * rr + PRO
    for j in range(3):
      i = i0 + j
      sl = (PRO + j) % 3
      slp = (PRO + j - 1) % 3
      gdrain(i, sl)
      sfire(sl)
      sdrain(slp)

      @pl.when(i + PRO < NCHUNK)
      def _():
        fire(i + PRO, slp)

    return carry

  lax.fori_loop(0, NRND, rounds, 0)
  sdrain((NCHUNK - 1) % 3)
  plsc.subcore_barrier()

  for j in range(RNDS):
    k = s + NS * j

    @pl.when(k < RCH)
    def _():
      pltpu.async_copy(acc.at[pl.ds(k * EB, EB)],
                       out_hbm.at[c, pl.ds(k * EB, EB)], semw)

  for j in range(RNDS):
    k = s + NS * j

    @pl.when(k < RCH)
    def _():
      pltpu.make_async_copy(acc.at[pl.ds(k * EB, EB)],
                            out_hbm.at[c, pl.ds(k * EB, EB)], semw).wait()


_sc_agg = pl.kernel(
    _sc_agg_body,
    out_type=jax.ShapeDtypeStruct((NC, N, D), jnp.float32),
    mesh=_MESH,
    scratch_types=[
        pltpu.VMEM_SHARED((N, D), jnp.float32),
        pltpu.VMEM((EPW,), jnp.int32),
        pltpu.VMEM((NSLOT, EB), jnp.int32),
        pltpu.VMEM((NSLOT, EB, D), jnp.float32),
        pltpu.SemaphoreType.DMA((NSLOT,)),
        pltpu.SemaphoreType.DMA((NSLOT,)),
        pltpu.SemaphoreType.DMA((NSLOT,)),
        pltpu.SemaphoreType.DMA,
    ],
    name="sc_gcn_aggregate",
)


DSLOT = 5


def _sc_deg_body(dst_hbm, out_hbm, acc, idxd, ones_v, stage_v, semw, sems):
  c = lax.axis_index("c")
  s = lax.axis_index("s")
  wid = s * NC + c

  for j in range(EB // 16):
    ones_v[pl.ds(j * 16, 16)] = jnp.ones((16,), jnp.float32)

  pltpu.sync_copy(dst_hbm.at[wid], idxd)

  for j in range(RNDS):
    k = s + NS * j

    @pl.when(k < RCH)
    def _():
      pltpu.async_copy(ones_v, acc.at[pl.ds(k * EB, EB)], semw)

  for j in range(RNDS):
    k = s + NS * j

    @pl.when(k < RCH)
    def _():
      pltpu.make_async_copy(ones_v, acc.at[pl.ds(k * EB, EB)], semw).wait()

  plsc.subcore_barrier()

  def rounds(rr, carry):
    for j in range(DSLOT):
      i = DSLOT * rr + j

      @pl.when(rr > 0)
      def _():
        pltpu.make_async_copy(ones_v, acc.at[idxd.at[i - DSLOT]],
                              sems.at[j]).wait()

      pltpu.async_copy(ones_v, acc.at[idxd.at[i]], sems.at[j], add=True)
    return carry

  lax.fori_loop(0, NCHUNK // DSLOT, rounds, 0)
  for j in range(DSLOT):
    i = NCHUNK - DSLOT + j
    pltpu.make_async_copy(ones_v, acc.at[idxd.at[i]], sems.at[j]).wait()

  plsc.subcore_barrier()

  for j in range(RNDS):
    k = s + NS * j

    @pl.when(k < RCH)
    def _():
      pltpu.async_copy(acc.at[pl.ds(k * EB, EB)], stage_v.at[j], semw)

  for j in range(RNDS):
    k = s + NS * j

    @pl.when(k < RCH)
    def _():
      pltpu.make_async_copy(acc.at[pl.ds(k * EB, EB)], stage_v.at[j],
                            semw).wait()
      pltpu.async_copy(stage_v.at[j],
                       out_hbm.at[pl.ds(c * N + k * EB, EB)], semw)

  for j in range(RNDS):
    k = s + NS * j

    @pl.when(k < RCH)
    def _():
      pltpu.make_async_copy(stage_v.at[j],
                            out_hbm.at[pl.ds(c * N + k * EB, EB)], semw).wait()


_sc_deg = pl.kernel(
    _sc_deg_body,
    out_type=jax.ShapeDtypeStruct((NC * N,), jnp.float32),
    mesh=_MESH,
    scratch_types=[
        pltpu.VMEM_SHARED((N,), jnp.float32),
        pltpu.VMEM((NCHUNK, EB), jnp.int32),
        pltpu.VMEM((EB,), jnp.float32),
        pltpu.VMEM((RNDS, EB), jnp.float32),
        pltpu.SemaphoreType.DMA,
        pltpu.SemaphoreType.DMA((DSLOT,)),
    ],
    name="sc_gcn_degree",
)


RBK = 2000
GRID = N // RBK


def _dis(pdegT):
  return lax.rsqrt(pdegT[:, 0:1] + pdegT[:, 1:2] - 1.0)


def _tc0_body(x_ref, w_ref, pdegT_ref, g_ref):
  dis = _dis(pdegT_ref[...])
  g_ref[...] = jnp.dot(x_ref[...], w_ref[...],
                       preferred_element_type=jnp.float32) * dis


def _tcA_body(p_ref, g_ref, pdegT_ref, w_ref, s_ref, t_ref, h_ref, gn_ref):
  dis = _dis(pdegT_ref[...])
  z = (p_ref[0] + p_ref[1] - g_ref[...]) * dis
  h = jnp.maximum(z * s_ref[...] + t_ref[...], 0.0)
  h_ref[...] = h
  gn_ref[...] = jnp.dot(h, w_ref[...],
                        preferred_element_type=jnp.float32) * dis


def _tcB_body(p_ref, g_ref, pdegT_ref, w_ref, s_ref, t_ref, res_ref, gn_ref):
  dis = _dis(pdegT_ref[...])
  z = (p_ref[0] + p_ref[1] - g_ref[...]) * dis
  h = jnp.maximum(z * s_ref[...] + t_ref[...], 0.0) + res_ref[...]
  gn_ref[...] = jnp.dot(h, w_ref[...],
                        preferred_element_type=jnp.float32) * dis


def _tcF_body(p_ref, g_ref, pdegT_ref, b_ref, o_ref):
  dis = _dis(pdegT_ref[...])
  o_ref[...] = (p_ref[0] + p_ref[1] - g_ref[...]) * dis + b_ref[...]


_row_spec = pl.BlockSpec((RBK, D), lambda i: (i, 0))
_p_spec = pl.BlockSpec((NC, RBK, D), lambda i: (0, i, 0))
_w_spec = pl.BlockSpec((D, D), lambda i: (0, 0))
_v_spec = pl.BlockSpec((1, D), lambda i: (0, 0))
_pdegT_spec = pl.BlockSpec((RBK, NC), lambda i: (i, 0))
_row_out = jax.ShapeDtypeStruct((N, D), jnp.float32)

_tc0 = pl.pallas_call(
    _tc0_body, grid=(GRID,),
    in_specs=[_row_spec, _w_spec, _pdegT_spec],
    out_specs=_row_spec, out_shape=_row_out)

_tcA = pl.pallas_call(
    _tcA_body, grid=(GRID,),
    in_specs=[_p_spec, _row_spec, _pdegT_spec, _w_spec, _v_spec, _v_spec],
    out_specs=(_row_spec, _row_spec), out_shape=(_row_out, _row_out))

_tcB = pl.pallas_call(
    _tcB_body, grid=(GRID,),
    in_specs=[_p_spec, _row_spec, _pdegT_spec, _w_spec, _v_spec, _v_spec,
              _row_spec],
    out_specs=_row_spec, out_shape=_row_out)

_tcF = pl.pallas_call(
    _tcF_body, grid=(GRID,),
    in_specs=[_p_spec, _row_spec, _pdegT_spec, _v_spec],
    out_specs=_row_spec, out_shape=_row_out)


def kernel(x, edge_index, W0, b0, W1, b1, W2, b2,
           bn0_g, bn0_b, bn0_m, bn0_v, bn1_g, bn1_b, bn1_m, bn1_v):
  src = edge_index[0].astype(jnp.int32)
  dst = edge_index[1].astype(jnp.int32)
  src3 = src.reshape(NW, EPW)
  dst3 = dst.reshape(NW, NCHUNK, EB)

  S0 = (bn0_g * lax.rsqrt(bn0_v + EPS)).reshape(1, D)
  T0 = ((b0 - bn0_m) * S0[0] + bn0_b).reshape(1, D)
  S1 = (bn1_g * lax.rsqrt(bn1_v + EPS)).reshape(1, D)
  T1 = ((b1 - bn1_m) * S1[0] + bn1_b).reshape(1, D)
  b2r = b2.reshape(1, D)

  pdegT = _sc_deg(dst3).reshape(NC, N).T

  g0 = _tc0(x, W0, pdegT)
  p0 = _sc_agg(g0, src3, dst3)
  h0, g1 = _tcA(p0, g0, pdegT, W1, S0, T0)
  p1 = _sc_agg(g1, src3, dst3)
  g2 = _tcB(p1, g1, pdegT, W2, S1, T1, h0)
  p2 = _sc_agg(g2, src3, dst3)
  return _tcF(p2, g2, pdegT, b2r)
